# baseline (speedup 1.0000x reference)
import jax
import jax.numpy as jnp
from jax import lax
from jax.experimental import pallas as pl
from jax.experimental.pallas import tpu as pltpu
from jax.experimental.pallas import tpu_sc as plsc

S, B, K, D = 8, 1024, 64, 32
NC, NS, L = 2, 16, 16
NW = NC * NS
ROWS = S * B
ROWS_PER_W = ROWS // NW
CHUNK = 128
NCHUNK = ROWS_PER_W // CHUNK


def _sc_body(p_hbm, mus_hbm, sig_hbm, z_hbm, out_hbm,
             z_buf, idx_buf, mu_buf, sg_buf, p_buf, sem0, sem1):
  wid = lax.axis_index("s") * NC + lax.axis_index("c")
  lane = lax.iota(jnp.int32, L)

  for c in range(NCHUNK):
    base = wid * ROWS_PER_W + c * CHUNK

    pltpu.sync_copy(z_hbm.at[pl.ds(base, CHUNK)], z_buf)
    for v in range(CHUNK // L):
      sl = pl.ds(v * L, L)
      idx_buf[sl] = (base + v * L + lane) * K + z_buf[sl]

    mu_cp = pltpu.async_copy(mus_hbm.at[idx_buf], mu_buf, sem0)
    sg_cp = pltpu.async_copy(sig_hbm.at[idx_buf], sg_buf, sem1)
    pltpu.sync_copy(p_hbm.at[pl.ds(base, CHUNK)], p_buf)
    mu_cp.wait()
    sg_cp.wait()

    def fma_row(i, _):
      for j in range(D // L):
        sj = pl.ds(j * L, L)
        mu_buf[i, sj] = mu_buf[i, sj] + sg_buf[i, sj] * p_buf[i, sj]
      return 0

    lax.fori_loop(0, CHUNK, fma_row, 0, unroll=2)

    pltpu.sync_copy(mu_buf, out_hbm.at[pl.ds(base, CHUNK)])


@jax.jit
def _sample_point_sc(p2, mus2, sig2, zf):
  mesh = plsc.VectorSubcoreMesh(core_axis_name="c", subcore_axis_name="s",
                                num_cores=NC, num_subcores=NS)
  run = pl.kernel(
      _sc_body,
      out_type=jax.ShapeDtypeStruct((ROWS, D), jnp.float32),
      mesh=mesh,
      scratch_types=[
          pltpu.VMEM((CHUNK,), jnp.int32),
          pltpu.VMEM((CHUNK,), jnp.int32),
          pltpu.VMEM((CHUNK, D), jnp.float32),
          pltpu.VMEM((CHUNK, D), jnp.float32),
          pltpu.VMEM((CHUNK, D), jnp.float32),
          pltpu.SemaphoreType.DMA,
          pltpu.SemaphoreType.DMA,
      ],
      compiler_params=pltpu.CompilerParams(use_tc_tiling_on_sc=False),
  )
  return run(p2, mus2, sig2, zf)


def kernel(p, mus, sigmas, z):
  p2 = p.reshape(ROWS, D)
  mus2 = mus.reshape(ROWS * K, D)
  sig2 = sigmas.reshape(ROWS * K, D)
  zf = z.reshape(ROWS).astype(jnp.int32)
  out = _sample_point_sc(p2, mus2, sig2, zf)
  return out.reshape(S, B, D)


# speedup vs baseline: 1.5241x; 1.5241x over previous
import jax
import jax.numpy as jnp
from jax.experimental import pallas as pl
from jax.experimental.pallas import tpu as pltpu

S, B, K, D = 8, 1024, 64, 32


def _select_body(z_ref, mu_ref, sg_ref, p_ref, out_ref, acc_sg):
  k = pl.program_id(1)
  zrow = z_ref[0, 0, :]
  mask = jnp.broadcast_to((zrow == k)[None, :], (D, B))
  mu = mu_ref[0, 0]
  sg = sg_ref[0, 0]

  @pl.when(k == 0)
  def _init():
    out_ref[0] = jnp.where(mask, mu, 0.0)
    acc_sg[...] = jnp.where(mask, sg, 0.0)

  @pl.when(k > 0)
  def _acc():
    out_ref[0] = jnp.where(mask, mu, out_ref[0])
    acc_sg[...] = jnp.where(mask, sg, acc_sg[...])

  @pl.when(k == K - 1)
  def _fin():
    out_ref[0] = out_ref[0] + acc_sg[...] * p_ref[0]


@jax.jit
def _sample_point_tc(mus_t, sig_t, p_t, z3):
  return pl.pallas_call(
      _select_body,
      grid=(S, K),
      in_specs=[
          pl.BlockSpec((1, 1, B), lambda s, k: (s, 0, 0)),
          pl.BlockSpec((1, 1, D, B), lambda s, k: (s, k, 0, 0)),
          pl.BlockSpec((1, 1, D, B), lambda s, k: (s, k, 0, 0)),
          pl.BlockSpec((1, D, B), lambda s, k: (s, 0, 0)),
      ],
      out_specs=pl.BlockSpec((1, D, B), lambda s, k: (s, 0, 0)),
      out_shape=jax.ShapeDtypeStruct((S, D, B), jnp.float32),
      scratch_shapes=[pltpu.VMEM((D, B), jnp.float32)],
      compiler_params=pltpu.CompilerParams(
          dimension_semantics=("arbitrary", "arbitrary")),
  )(z3, mus_t, sig_t, p_t)


def kernel(p, mus, sigmas, z):
  mus_t = mus.transpose(0, 2, 3, 1)
  sig_t = sigmas.transpose(0, 2, 3, 1)
  p_t = p.transpose(0, 2, 1)
  z3 = z.reshape(S, 1, B).astype(jnp.int32)
  out_t = _sample_point_tc(mus_t, sig_t, p_t, z3)
  return out_t.transpose(0, 2, 1)


# speedup vs baseline: 6.9957x; 4.5900x over previous
import jax
import jax.numpy as jnp
from jax.experimental import pallas as pl
from jax.experimental.pallas import tpu as pltpu

S, B, K, D = 8, 1024, 64, 32
KB = 8
NKB = K // KB


def _select_body(z_ref, mu_ref, sg_ref, p_ref, out_ref, acc_sg, zb):
  kb = pl.program_id(1)

  @pl.when(kb == 0)
  def _bcast():
    zb[...] = jnp.broadcast_to(z_ref[0, 0, :][None, :], (D, B))

  zt = zb[...]
  acc_mu = out_ref[0]
  acc_s = acc_sg[...]
  for kk in range(KB):
    m = zt == (kb * KB + kk)
    acc_mu = jnp.where(m, mu_ref[0, kk], acc_mu)
    acc_s = jnp.where(m, sg_ref[0, kk], acc_s)

  @pl.when(kb < NKB - 1)
  def _store():
    out_ref[0] = acc_mu
    acc_sg[...] = acc_s

  @pl.when(kb == NKB - 1)
  def _fin():
    out_ref[0] = acc_mu + acc_s * p_ref[0]


@jax.jit
def _sample_point_tc(mus_t, sig_t, p_t, z3):
  return pl.pallas_call(
      _select_body,
      grid=(S, NKB),
      in_specs=[
          pl.BlockSpec((1, 1, B), lambda s, kb: (s, 0, 0)),
          pl.BlockSpec((1, KB, D, B), lambda s, kb: (s, kb, 0, 0)),
          pl.BlockSpec((1, KB, D, B), lambda s, kb: (s, kb, 0, 0)),
          pl.BlockSpec((1, D, B), lambda s, kb: (s, 0, 0)),
      ],
      out_specs=pl.BlockSpec((1, D, B), lambda s, kb: (s, 0, 0)),
      out_shape=jax.ShapeDtypeStruct((S, D, B), jnp.float32),
      scratch_shapes=[pltpu.VMEM((D, B), jnp.float32),
                      pltpu.VMEM((D, B), jnp.int32)],
      compiler_params=pltpu.CompilerParams(
          dimension_semantics=("arbitrary", "arbitrary")),
  )(z3, mus_t, sig_t, p_t)


def kernel(p, mus, sigmas, z):
  mus_t = mus.transpose(0, 2, 3, 1)
  sig_t = sigmas.transpose(0, 2, 3, 1)
  p_t = p.transpose(0, 2, 1)
  z3 = z.reshape(S, 1, B).astype(jnp.int32)
  out_t = _sample_point_tc(mus_t, sig_t, p_t, z3)
  return out_t.transpose(0, 2, 1)


# speedup vs baseline: 9.4905x; 1.3566x over previous
import jax
import jax.numpy as jnp
from jax.experimental import pallas as pl
from jax.experimental.pallas import tpu as pltpu

S, B, K, D = 8, 1024, 64, 32
KB = 16
NKB = K // KB


def _select_body(z_ref, mu_ref, sg_ref, p_ref, out_ref, acc_sg, zb):
  kb = pl.program_id(1)

  @pl.when(kb == 0)
  def _bcast():
    zb[...] = jnp.broadcast_to(z_ref[0, 0, :][None, :], (D, B))

  zt = zb[...]
  acc_mu = out_ref[0]
  acc_s = acc_sg[...]
  for kk in range(KB):
    m = zt == (kb * KB + kk)
    acc_mu = jnp.where(m, mu_ref[0, kk], acc_mu)
    acc_s = jnp.where(m, sg_ref[0, kk], acc_s)

  @pl.when(kb < NKB - 1)
  def _store():
    out_ref[0] = acc_mu
    acc_sg[...] = acc_s

  @pl.when(kb == NKB - 1)
  def _fin():
    out_ref[0] = acc_mu + acc_s * p_ref[0]


@jax.jit
def _sample_point_tc(mus_t, sig_t, p_t, z3):
  return pl.pallas_call(
      _select_body,
      grid=(S, NKB),
      in_specs=[
          pl.BlockSpec((1, 1, B), lambda s, kb: (s, 0, 0)),
          pl.BlockSpec((1, KB, D, B), lambda s, kb: (s, kb, 0, 0)),
          pl.BlockSpec((1, KB, D, B), lambda s, kb: (s, kb, 0, 0)),
          pl.BlockSpec((1, D, B), lambda s, kb: (s, 0, 0)),
      ],
      out_specs=pl.BlockSpec((1, D, B), lambda s, kb: (s, 0, 0)),
      out_shape=jax.ShapeDtypeStruct((S, D, B), jnp.float32),
      scratch_shapes=[pltpu.VMEM((D, B), jnp.float32),
                      pltpu.VMEM((D, B), jnp.int32)],
      compiler_params=pltpu.CompilerParams(
          dimension_semantics=("arbitrary", "arbitrary")),
  )(z3, mus_t, sig_t, p_t)


def kernel(p, mus, sigmas, z):
  mus_t = mus.transpose(0, 2, 3, 1)
  sig_t = sigmas.transpose(0, 2, 3, 1)
  p_t = p.transpose(0, 2, 1)
  z3 = z.reshape(S, 1, B).astype(jnp.int32)
  out_t = _sample_point_tc(mus_t, sig_t, p_t, z3)
  return out_t.transpose(0, 2, 1)


# speedup vs baseline: 10.6231x; 1.1193x over previous
import jax
import jax.numpy as jnp
from jax.experimental import pallas as pl
from jax.experimental.pallas import tpu as pltpu

S, B, K, D = 8, 1024, 64, 32
KB = 32
NKB = K // KB


def _select_body(z_ref, mu_ref, sg_ref, p_ref, out_ref, acc_sg, zb):
  kb = pl.program_id(1)

  @pl.when(kb == 0)
  def _bcast():
    zb[...] = jnp.broadcast_to(z_ref[0, 0, :][None, :], (D, B))

  zt = zb[...]
  acc_mu = out_ref[0]
  acc_s = acc_sg[...]
  for kk in range(KB):
    m = zt == (kb * KB + kk)
    acc_mu = jnp.where(m, mu_ref[0, kk], acc_mu)
    acc_s = jnp.where(m, sg_ref[0, kk], acc_s)

  @pl.when(kb < NKB - 1)
  def _store():
    out_ref[0] = acc_mu
    acc_sg[...] = acc_s

  @pl.when(kb == NKB - 1)
  def _fin():
    out_ref[0] = acc_mu + acc_s * p_ref[0]


@jax.jit
def _sample_point_tc(mus_t, sig_t, p_t, z3):
  return pl.pallas_call(
      _select_body,
      grid=(S, NKB),
      in_specs=[
          pl.BlockSpec((1, 1, B), lambda s, kb: (s, 0, 0)),
          pl.BlockSpec((1, KB, D, B), lambda s, kb: (s, kb, 0, 0)),
          pl.BlockSpec((1, KB, D, B), lambda s, kb: (s, kb, 0, 0)),
          pl.BlockSpec((1, D, B), lambda s, kb: (s, 0, 0)),
      ],
      out_specs=pl.BlockSpec((1, D, B), lambda s, kb: (s, 0, 0)),
      out_shape=jax.ShapeDtypeStruct((S, D, B), jnp.float32),
      scratch_shapes=[pltpu.VMEM((D, B), jnp.float32),
                      pltpu.VMEM((D, B), jnp.int32)],
      compiler_params=pltpu.CompilerParams(
          dimension_semantics=("arbitrary", "arbitrary")),
  )(z3, mus_t, sig_t, p_t)


def kernel(p, mus, sigmas, z):
  mus_t = mus.transpose(0, 2, 3, 1)
  sig_t = sigmas.transpose(0, 2, 3, 1)
  p_t = p.transpose(0, 2, 1)
  z3 = z.reshape(S, 1, B).astype(jnp.int32)
  out_t = _sample_point_tc(mus_t, sig_t, p_t, z3)
  return out_t.transpose(0, 2, 1)
